# baseline (speedup 1.0000x reference)
import functools

import jax
import jax.numpy as jnp
from jax import lax
from jax.experimental import pallas as pl
from jax.experimental.pallas import tpu as pltpu
from jax.experimental.pallas import tpu_sc as plsc

BATCH = 16384
VOCAB = 1000000
DIM = 64

_INFO = plsc.get_sparse_core_info()
_NC, _NS = _INFO.num_cores, _INFO.num_subcores
_NW = _NC * _NS
_STRIPE = 31232
_WIN = 512
_NWIN = _STRIPE // _WIN
_SWEPT = 999936
_GRP = 64
_TRASH = BATCH

_MESH = plsc.VectorSubcoreMesh(core_axis_name="c", subcore_axis_name="s")


@functools.partial(
    pl.kernel,
    mesh=_MESH,
    out_type=jax.ShapeDtypeStruct((BATCH + 16, 2 * DIM), jnp.float32),
    scratch_types=[
        pltpu.VMEM((BATCH,), jnp.int32),
        pltpu.VMEM((BATCH + 16,), jnp.int32),
        pltpu.VMEM((BATCH + 16,), jnp.int32),
        pltpu.VMEM((2, DIM, _WIN), jnp.float32),
        pltpu.VMEM((_GRP, 2 * DIM), jnp.float32),
        pltpu.VMEM((_GRP,), jnp.int32),
        pltpu.VMEM((16,), jnp.int32),
        pltpu.VMEM((16,), jnp.int32),
        pltpu.SemaphoreType.DMA,
        pltpu.SemaphoreType.DMA,
    ],
    compiler_params=pltpu.CompilerParams(needs_layout_passes=False),
)
def _sweep_gather(idx_hbm, tt_hbm, inter_hbm,
                  idxall, mr, mb, chv, rowbuf, sidx, tr, tb, sem, wsem):
    wid = lax.axis_index("s") * _NC + lax.axis_index("c")
    lo = wid * _STRIPE
    hi = jnp.where(wid == _NW - 1, jnp.int32(_SWEPT), lo + _STRIPE)
    lanes = lax.iota(jnp.int32, 16)
    nwin = jnp.where(wid == _NW - 1, _NWIN + 1, _NWIN)

    def win_copy(k):
        ws = pl.multiple_of(lo + k * _WIN, _WIN)
        return pltpu.make_async_copy(
            tt_hbm.at[:, pl.ds(ws, _WIN)], chv.at[k & 1], wsem)

    win_copy(jnp.int32(0)).start()
    pltpu.sync_copy(idx_hbm, idxall)

    def scan(t, cnt):
        v = idxall[pl.ds(t * 16, 16)]
        m = (v >= lo) & (v < hi)
        pos = cnt + plsc.cumsum(m.astype(jnp.int32)) - 1
        plsc.store_scatter(mr, [pos], v, mask=m)
        plsc.store_scatter(mb, [pos], lanes + t * 16, mask=m)
        return cnt + plsc.all_reduce_population_count(m)[0]

    cnt = lax.fori_loop(0, BATCH // 16, scan, jnp.int32(0))
    plsc.store_scatter(mr, [cnt + lanes], jnp.full((16,), -1, jnp.int32))
    nvec = (cnt + 15) // 16

    def init_group():
        for q in range(_GRP // 16):
            sidx[pl.ds(q * 16, 16)] = lanes + jnp.int32(_TRASH)

    init_group()

    def flush(gsel_unused):
        pltpu.async_copy(rowbuf, inter_hbm.at[sidx], sem).wait()
        init_group()

    def extract_window(ws, slot, g):
        def per_vec(j, g):
            r16 = mr[pl.ds(j * 16, 16)]
            m = (r16 >= ws) & (r16 < ws + _WIN)
            n = plsc.all_reduce_population_count(m)[0]

            def matches(g):
                b16 = mb[pl.ds(j * 16, 16)]
                plsc.store_compressed(tr.at[pl.ds(0, 16)], r16, mask=m)
                plsc.store_compressed(tb.at[pl.ds(0, 16)], b16, mask=m)
                return lax.fori_loop(0, n, per_match, g)

            def per_match(i, g):
                riv = plsc.load_gather(tr, [jnp.full((16,), i, jnp.int32)])
                biv = plsc.load_gather(tb, [jnp.full((16,), i, jnp.int32)])
                col = riv - ws
                gi = g % _GRP
                for k in range(DIM // 16):
                    vals = plsc.load_gather(
                        chv.at[slot], [lanes + k * 16, col])
                    rowbuf[gi, pl.ds(k * 16, 16)] = vals
                plsc.store_scatter(
                    sidx, [jnp.full((16,), gi, jnp.int32)], biv,
                    mask=(lanes == 0))
                g = g + 1

                @pl.when(g % _GRP == 0)
                def _():
                    flush(0)

                return g

            return lax.cond(n > 0, matches, lambda g: g, g)

        return lax.fori_loop(0, nvec, per_vec, g)

    def per_window(k, g):
        ws = pl.multiple_of(lo + k * _WIN, _WIN)
        win_copy(k).wait()

        @pl.when(k + 1 < nwin)
        def _():
            win_copy(k + 1).start()

        return extract_window(ws, k & 1, g)

    lax.fori_loop(0, nwin, per_window, jnp.int32(0))
    flush(0)


def kernel(context, table, empty_context):
    idx = context.astype(jnp.int32)
    inter = _sweep_gather(idx, table.T)
    out = inter[:BATCH, :DIM]
    tail = idx - jnp.int32(_SWEPT)
    onehot = (tail[:, None] == jnp.arange(DIM, dtype=jnp.int32)[None, :])
    fix = onehot.astype(jnp.float32) @ table[_SWEPT:, :]
    fix = lax.optimization_barrier(fix)
    out = jnp.where((tail >= 0)[:, None], fix, out)
    return out[:, None, :]


# speedup vs baseline: 1.0450x; 1.0450x over previous
import functools

import jax
import jax.numpy as jnp
from jax import lax
from jax.experimental import pallas as pl
from jax.experimental.pallas import tpu as pltpu
from jax.experimental.pallas import tpu_sc as plsc

BATCH = 16384
VOCAB = 1000000
DIM = 64

_INFO = plsc.get_sparse_core_info()
_NC, _NS = _INFO.num_cores, _INFO.num_subcores
_NW = _NC * _NS
_STRIPE = 31232
_WIN = 512
_NWIN = _STRIPE // _WIN
_SWEPT = 999936
_GRP = 32
_TRASH = BATCH

_MESH = plsc.VectorSubcoreMesh(core_axis_name="c", subcore_axis_name="s")


@functools.partial(
    pl.kernel,
    mesh=_MESH,
    out_type=jax.ShapeDtypeStruct((BATCH + 16, 2 * DIM), jnp.float32),
    scratch_types=[
        pltpu.VMEM((BATCH,), jnp.int32),
        pltpu.VMEM((BATCH + 16,), jnp.int32),
        pltpu.VMEM((BATCH + 16,), jnp.int32),
        pltpu.VMEM((2, DIM, _WIN), jnp.float32),
        pltpu.VMEM((_GRP, 2 * DIM), jnp.float32),
        pltpu.VMEM((_GRP,), jnp.int32),
        pltpu.VMEM((16,), jnp.int32),
        pltpu.VMEM((16,), jnp.int32),
        pltpu.VMEM((DIM, DIM), jnp.float32),
        pltpu.SemaphoreType.DMA,
        pltpu.SemaphoreType.DMA,
    ],
    compiler_params=pltpu.CompilerParams(needs_layout_passes=False),
)
def _sweep_gather(idx_hbm, tt_hbm, slab_hbm, inter_hbm,
                  idxall, mr, mb, chv, rowbuf, sidx, tr, tb, slabv,
                  sem, wsem):
    wid = lax.axis_index("s") * _NC + lax.axis_index("c")
    lo = wid * _STRIPE
    hi = jnp.where(wid == _NW - 1, jnp.int32(VOCAB), lo + _STRIPE)
    lanes = lax.iota(jnp.int32, 16)
    nwin = jnp.where(wid == _NW - 1, _NWIN + 1, _NWIN)

    def win_copy(k):
        ws = pl.multiple_of(lo + k * _WIN, _WIN)
        return pltpu.make_async_copy(
            tt_hbm.at[:, pl.ds(ws, _WIN)], chv.at[k & 1], wsem)

    win_copy(jnp.int32(0)).start()
    pltpu.sync_copy(idx_hbm, idxall)

    def scan(t, cnt):
        v = idxall[pl.ds(t * 16, 16)]
        m = (v >= lo) & (v < hi)
        pos = cnt + plsc.cumsum(m.astype(jnp.int32)) - 1
        plsc.store_scatter(mr, [pos], v, mask=m)
        plsc.store_scatter(mb, [pos], lanes + t * 16, mask=m)
        return cnt + plsc.all_reduce_population_count(m)[0]

    cnt = lax.fori_loop(0, BATCH // 16, scan, jnp.int32(0))
    plsc.store_scatter(mr, [cnt + lanes], jnp.full((16,), -1, jnp.int32))
    nvec = (cnt + 15) // 16

    def init_group():
        for q in range(_GRP // 16):
            sidx[pl.ds(q * 16, 16)] = lanes + jnp.int32(_TRASH)

    init_group()

    def flush(gsel_unused):
        pltpu.async_copy(rowbuf, inter_hbm.at[sidx], sem).wait()
        init_group()

    def extract_window(ws, buf, g):
        def per_vec(j, g):
            r16 = mr[pl.ds(j * 16, 16)]
            m = (r16 >= ws) & (r16 < ws + _WIN)
            n = plsc.all_reduce_population_count(m)[0]

            def matches(g):
                b16 = mb[pl.ds(j * 16, 16)]
                plsc.store_compressed(tr.at[pl.ds(0, 16)], r16, mask=m)
                plsc.store_compressed(tb.at[pl.ds(0, 16)], b16, mask=m)
                return lax.fori_loop(0, n, per_match, g)

            def per_match(i, g):
                riv = plsc.load_gather(tr, [jnp.full((16,), i, jnp.int32)])
                biv = plsc.load_gather(tb, [jnp.full((16,), i, jnp.int32)])
                col = riv - ws
                gi = g % _GRP
                for k in range(DIM // 16):
                    vals = plsc.load_gather(buf, [lanes + k * 16, col])
                    rowbuf[gi, pl.ds(k * 16, 16)] = vals
                plsc.store_scatter(
                    sidx, [jnp.full((16,), gi, jnp.int32)], biv,
                    mask=(lanes == 0))
                g = g + 1

                @pl.when(g % _GRP == 0)
                def _():
                    flush(0)

                return g

            return lax.cond(n > 0, matches, lambda g: g, g)

        return lax.fori_loop(0, nvec, per_vec, g)

    def per_window(k, g):
        ws = pl.multiple_of(lo + k * _WIN, _WIN)
        win_copy(k).wait()

        @pl.when(k + 1 < nwin)
        def _():
            win_copy(k + 1).start()

        return extract_window(ws, chv.at[k & 1], g)

    g = lax.fori_loop(0, nwin, per_window, jnp.int32(0))

    @pl.when(wid == _NW - 1)
    def _():
        pltpu.sync_copy(slab_hbm, slabv)
        extract_window(jnp.int32(_SWEPT), slabv.at[pl.ds(0, DIM)], g)

    flush(0)


def kernel(context, table, empty_context):
    idx = context.astype(jnp.int32)
    tt = table.T
    inter = _sweep_gather(idx, tt, tt[:, _SWEPT:])
    return inter[:BATCH, :DIM][:, None, :]
